# baseline (speedup 1.0000x reference)
import functools

import jax
import jax.numpy as jnp
from jax import lax
from jax.experimental import pallas as pl
from jax.experimental.pallas import tpu as pltpu
from jax.experimental.pallas import tpu_sc as plsc

NC = 2
NS = 16
NW = NC * NS

B = 4
S = 8192
D = 1024
ROWS_PER_W = S // NW
CHUNK = 32

STREAM_ROWS = 224
SPMEM_ROWS = ROWS_PER_W - STREAM_ROWS
N_ST = STREAM_ROWS // CHUNK
N_SP = SPMEM_ROWS // CHUNK


def _make_sc_copy():
    mesh = plsc.VectorSubcoreMesh(core_axis_name="c", subcore_axis_name="s")

    @functools.partial(
        pl.kernel,
        out_type=jax.ShapeDtypeStruct((B, S, D), jnp.float32),
        mesh=mesh,
        scratch_types=[
            pltpu.VMEM((CHUNK, D), jnp.float32),
            pltpu.VMEM((CHUNK, D), jnp.float32),
            pltpu.VMEM_SHARED((NS, 2, CHUNK, D), jnp.float32),
            pltpu.SemaphoreType.DMA,
            pltpu.SemaphoreType.DMA,
            pltpu.SemaphoreType.DMA,
            pltpu.SemaphoreType.DMA,
            pltpu.SemaphoreType.DMA,
            pltpu.SemaphoreType.DMA,
            pltpu.SemaphoreType.DMA,
            pltpu.SemaphoreType.DMA,
        ],
    )
    def body(table_hbm, out_hbm, buf0, buf1, shb,
             isem0, isem1, osem0, osem1,
             s_isem0, s_isem1, s_osem0, s_osem1):
        cid = lax.axis_index("c")
        sid = lax.axis_index("s")
        wid = sid * NC + cid
        base = wid * ROWS_PER_W

        class Pipe:
            def __init__(self, r_base, n, bufs, isems, osems):
                self.r_base, self.n = r_base, n
                self.bufs, self.isems, self.osems = bufs, isems, osems
                self.in_h = [None] * n
                self.out_h = [[] for _ in range(n)]
                self.step = 0

            def start_in(self, i):
                r0 = self.r_base + i * CHUNK
                self.in_h[i] = pltpu.async_copy(
                    table_hbm.at[pl.ds(r0, CHUNK)],
                    self.bufs[i % 2], self.isems[i % 2])

            def advance(self):
                i = self.step
                if i >= self.n:
                    return False
                cur = i % 2
                self.in_h[i].wait()
                r0 = self.r_base + i * CHUNK
                for b in range(B):
                    self.out_h[i].append(pltpu.async_copy(
                        self.bufs[cur],
                        out_hbm.at[b, pl.ds(r0, CHUNK)],
                        self.osems[cur]))
                if i + 1 < self.n:
                    if i >= 1:
                        for h in self.out_h[i - 1]:
                            h.wait()
                    self.start_in(i + 1)
                self.step += 1
                return True

            def drain(self):
                for i in range(max(0, self.n - 2), self.n):
                    for h in self.out_h[i]:
                        h.wait()

        stream_pipe = Pipe(base, N_ST, (buf0, buf1),
                           (isem0, isem1), (osem0, osem1))
        spmem_pipe = Pipe(base + STREAM_ROWS, N_SP,
                          (shb.at[sid, 0], shb.at[sid, 1]),
                          (s_isem0, s_isem1), (s_osem0, s_osem1))

        stream_pipe.start_in(0)
        spmem_pipe.start_in(0)
        alive = True
        while alive:
            alive = False
            alive |= spmem_pipe.advance()
            alive |= stream_pipe.advance()
        spmem_pipe.drain()
        stream_pipe.drain()

    return body


_sc_copy = _make_sc_copy()


def kernel(x, pos_table):
    del x
    return _sc_copy(pos_table)


# speedup vs baseline: 1.0172x; 1.0172x over previous
import functools

import jax
import jax.numpy as jnp
from jax import lax
from jax.experimental import pallas as pl
from jax.experimental.pallas import tpu as pltpu
from jax.experimental.pallas import tpu_sc as plsc

NC = 2
NS = 16
NW = NC * NS

B = 4
S = 8192
D = 1024
ROWS_PER_W = S // NW
CHUNK = 32
N_CHUNKS = ROWS_PER_W // CHUNK
N_SP = 2
N_ST = N_CHUNKS - N_SP


def _make_sc_copy():
    mesh = plsc.VectorSubcoreMesh(core_axis_name="c", subcore_axis_name="s")

    @functools.partial(
        pl.kernel,
        out_type=jax.ShapeDtypeStruct((B, S, D), jnp.float32),
        mesh=mesh,
        scratch_types=[
            pltpu.VMEM((CHUNK, D), jnp.float32),
            pltpu.VMEM((CHUNK, D), jnp.float32),
            pltpu.VMEM_SHARED((NS, N_SP, CHUNK, D), jnp.float32),
            pltpu.SemaphoreType.DMA,
            pltpu.SemaphoreType.DMA,
            pltpu.SemaphoreType.DMA,
            pltpu.SemaphoreType.DMA,
            pltpu.SemaphoreType.DMA,
            pltpu.SemaphoreType.DMA,
        ],
    )
    def body(table_hbm, out_hbm, buf0, buf1, shb,
             isem0, isem1, osem0, osem1, csem, s_osem):
        cid = lax.axis_index("c")
        sid = lax.axis_index("s")
        wid = sid * NC + cid
        base = wid * ROWS_PER_W
        bufs = (buf0, buf1)
        isems = (isem0, isem1)
        osems = (osem0, osem1)

        order = list(range(N_ST, N_CHUNKS)) + list(range(N_ST))

        def start_in(slot, chunk):
            r0 = base + chunk * CHUNK
            return pltpu.async_copy(
                table_hbm.at[pl.ds(r0, CHUNK)], bufs[slot], isems[slot])

        in_h = {}
        sp_out_h = []
        st_out_h = {}

        in_h[0] = start_in(0, order[0])
        in_h[1] = start_in(1, order[1])

        for k, chunk in enumerate(order):
            slot = k % 2
            r0 = base + chunk * CHUNK
            in_h[k].wait()
            if chunk >= N_ST:
                sp = chunk - N_ST
                pltpu.async_copy(bufs[slot], shb.at[sid, sp], csem).wait()
                for b in range(B):
                    sp_out_h.append(pltpu.async_copy(
                        shb.at[sid, sp],
                        out_hbm.at[b, pl.ds(r0, CHUNK)], s_osem))
            else:
                st_out_h[k] = [
                    pltpu.async_copy(
                        bufs[slot],
                        out_hbm.at[b, pl.ds(r0, CHUNK)], osems[slot])
                    for b in range(B)
                ]
            if k + 2 < N_CHUNKS:
                if k - 2 in st_out_h:
                    for h in st_out_h.pop(k - 2):
                        h.wait()
                in_h[k + 2] = start_in(slot, order[k + 2])

        for hs in st_out_h.values():
            for h in hs:
                h.wait()
        for h in sp_out_h:
            h.wait()

    return body


_sc_copy = _make_sc_copy()


def kernel(x, pos_table):
    del x
    return _sc_copy(pos_table)
